# baseline (speedup 1.0000x reference)
import functools

import jax
import jax.numpy as jnp
from jax import lax
from jax.experimental import pallas as pl
from jax.experimental.pallas import tpu as pltpu
from jax.experimental.pallas import tpu_sc as plsc

B, N = 4, 4096
D = 768
H = 768
M = B * N
POS = 10240
NC, NS = 2, 16
NW = NC * NS
S = 4
MG = M // S
MPW = MG // NW
C = 32
NCHUNK = MPW // C
IPW = MPW * 2


@functools.cache
def _pe_gather_kernel():
    mesh = plsc.VectorSubcoreMesh(core_axis_name="c", subcore_axis_name="s")

    @functools.partial(
        pl.kernel,
        mesh=mesh,
        out_type=jax.ShapeDtypeStruct((MG, H), jnp.float32),
        scratch_types=[
            pltpu.VMEM((IPW,), jnp.int32),
            [pltpu.VMEM((2 * C, H), jnp.float32) for _ in range(2)],
            [pltpu.SemaphoreType.DMA for _ in range(2)],
            [pltpu.SemaphoreType.DMA for _ in range(2)],
        ],
    )
    def k(tab_hbm, idx_hbm, out_hbm, idxv, rows, gsem, ssem):
        wid = lax.axis_index("s") * NC + lax.axis_index("c")
        base = wid * MPW
        pltpu.sync_copy(idx_hbm.at[pl.ds(wid * IPW, IPW)], idxv)

        def gather_start(j):
            b = j % 2
            pltpu.async_copy(
                tab_hbm.at[idxv.at[pl.ds(j * 2 * C, 2 * C)]], rows[b], gsem[b]
            )

        def gather_wait(j):
            b = j % 2
            pltpu.make_async_copy(
                tab_hbm.at[idxv.at[pl.ds(0, 2 * C)]], rows[b], gsem[b]
            ).wait()

        def scatter_start(j):
            b = j % 2
            pltpu.async_copy(
                rows[b].at[pl.ds(0, C)], out_hbm.at[pl.ds(base + j * C, C)],
                ssem[b],
            )

        def scatter_wait(j):
            b = j % 2
            pltpu.make_async_copy(
                rows[b].at[pl.ds(0, C)], out_hbm.at[pl.ds(base, C)], ssem[b]
            ).wait()

        def add_rows(j):
            b = j % 2

            def add_row(r, c2):
                for c in range(H // 16):
                    sl = pl.ds(c * 16, 16)
                    rows[b][r, sl] = rows[b][r, sl] + rows[b][C + r, sl]
                return c2

            lax.fori_loop(0, C, add_row, 0)

        gather_start(0)
        for j in range(NCHUNK):
            gather_wait(j)
            if j + 1 < NCHUNK:
                if j >= 1:
                    scatter_wait(j - 1)
                gather_start(j + 1)
            add_rows(j)
            scatter_start(j)
        scatter_wait(NCHUNK - 2)
        scatter_wait(NCHUNK - 1)

    return k


def _pe_gather(table2, idx2g):
    return _pe_gather_kernel()(table2, idx2g)


BM = 1024
GB = MG // BM


def _mm_body(px_ref, w_ref, pe_ref, out_ref):
    pxn = 2.0 * px_ref[...] - 1.0
    acc = lax.dot_general(
        pxn,
        w_ref[...],
        (((1,), (1,)), ((), ())),
        preferred_element_type=jnp.float32,
        precision=lax.Precision.DEFAULT,
    )
    out_ref[...] = acc + pe_ref[...]


def _mm_body_acc(px_ref, w_ref, pe_ref, h_ref, out_ref):
    del h_ref
    _mm_body(px_ref, w_ref, pe_ref, out_ref)


def _mm_group(g, px, w, pe_g, h):
    out_spec = pl.BlockSpec((BM, H), lambda i, g=g: (g * GB + i, 0))
    in_specs = [
        pl.BlockSpec((BM, D), lambda i, g=g: (g * GB + i, 0)),
        pl.BlockSpec((H, D), lambda i: (0, 0)),
        pl.BlockSpec((BM, H), lambda i: (i, 0)),
    ]
    if g == 0:
        return pl.pallas_call(
            _mm_body,
            grid=(GB,),
            in_specs=in_specs,
            out_specs=out_spec,
            out_shape=jax.ShapeDtypeStruct((M, H), jnp.float32),
        )(px, w, pe_g)
    return pl.pallas_call(
        _mm_body_acc,
        grid=(GB,),
        in_specs=in_specs + [pl.BlockSpec(memory_space=pl.ANY)],
        out_specs=out_spec,
        out_shape=jax.ShapeDtypeStruct((M, H), jnp.float32),
        input_output_aliases={3: 0},
    )(px, w, pe_g, h)


def kernel(pixel_values, pixel_position_ids, padding_mask, W, pos_table):
    del padding_mask
    px = pixel_values.reshape(M, D)
    table2 = pos_table.reshape(2 * POS, H)
    ids = pixel_position_ids.reshape(M, 2)
    ix = ids[:, 0].reshape(M // C, C)
    iy = ids[:, 1].reshape(M // C, C) + POS
    idx2 = jnp.stack([ix, iy], axis=1).reshape(2 * M)
    pes = [
        _pe_gather(table2, lax.slice(idx2, (g * 2 * MG,), ((g + 1) * 2 * MG,)))
        for g in range(S)
    ]
    h = None
    for g in range(S):
        h = _mm_group(g, px, W, pes[g], h)
    return h.reshape(B, N, H)


# speedup vs baseline: 1.1780x; 1.1780x over previous
import functools

import jax
import jax.numpy as jnp
from jax import lax
from jax.experimental import pallas as pl
from jax.experimental.pallas import tpu as pltpu
from jax.experimental.pallas import tpu_sc as plsc

B, N = 4, 4096
D = 768
H = 768
M = B * N
POS = 10240
NC, NS = 2, 16
NW = NC * NS
C = 64
BM = 1024
GROUPS = (8192, 8192)


@functools.cache
def _pe_gather_kernel(mg):
    mpw = mg // NW
    nchunk = mpw // C
    ipw = mpw * 2
    mesh = plsc.VectorSubcoreMesh(core_axis_name="c", subcore_axis_name="s")

    @functools.partial(
        pl.kernel,
        mesh=mesh,
        out_type=jax.ShapeDtypeStruct((mg, H), jnp.float32),
        scratch_types=[
            pltpu.VMEM((ipw,), jnp.int32),
            pltpu.VMEM((2 * C, H), jnp.float32),
            pltpu.SemaphoreType.DMA,
        ],
    )
    def k(tab_hbm, idx_hbm, out_hbm, idxv, rows, sem):
        wid = lax.axis_index("s") * NC + lax.axis_index("c")
        pltpu.sync_copy(idx_hbm.at[pl.ds(wid * ipw, ipw)], idxv)

        def chunk(j, carry):
            pltpu.async_copy(
                tab_hbm.at[idxv.at[pl.ds(j * 2 * C, 2 * C)]], rows, sem
            ).wait()

            def add_row(r, c2):
                for c in range(H // 16):
                    sl = pl.ds(c * 16, 16)
                    rows[r, sl] = rows[r, sl] + rows[C + r, sl]
                return c2

            lax.fori_loop(0, C, add_row, 0)
            off = wid * mpw + j * C
            pltpu.sync_copy(rows.at[pl.ds(0, C)], out_hbm.at[pl.ds(off, C)])
            return carry

        lax.fori_loop(0, nchunk, chunk, 0)

    return k


def _mm_body(px_ref, w_ref, pe_ref, out_ref):
    pxn = 2.0 * px_ref[...] - 1.0
    acc = lax.dot_general(
        pxn,
        w_ref[...],
        (((1,), (1,)), ((), ())),
        preferred_element_type=jnp.float32,
        precision=lax.Precision.DEFAULT,
    )
    out_ref[...] = acc + pe_ref[...]


def _mm_body_acc(px_ref, w_ref, pe_ref, h_ref, out_ref):
    del h_ref
    _mm_body(px_ref, w_ref, pe_ref, out_ref)


def _mm_group(tok0, mg, px, w, pe_g, h):
    b0 = tok0 // BM
    out_spec = pl.BlockSpec((BM, H), lambda i, b0=b0: (b0 + i, 0))
    in_specs = [
        pl.BlockSpec((BM, D), lambda i, b0=b0: (b0 + i, 0)),
        pl.BlockSpec((H, D), lambda i: (0, 0)),
        pl.BlockSpec((BM, H), lambda i: (i, 0)),
    ]
    if h is None:
        return pl.pallas_call(
            _mm_body,
            grid=(mg // BM,),
            in_specs=in_specs,
            out_specs=out_spec,
            out_shape=jax.ShapeDtypeStruct((M, H), jnp.float32),
        )(px, w, pe_g)
    return pl.pallas_call(
        _mm_body_acc,
        grid=(mg // BM,),
        in_specs=in_specs + [pl.BlockSpec(memory_space=pl.ANY)],
        out_specs=out_spec,
        out_shape=jax.ShapeDtypeStruct((M, H), jnp.float32),
        input_output_aliases={3: 0},
    )(px, w, pe_g, h)


def kernel(pixel_values, pixel_position_ids, padding_mask, W, pos_table):
    del padding_mask
    px = pixel_values.reshape(M, D)
    table2 = pos_table.reshape(2 * POS, H)
    ids = pixel_position_ids.reshape(M, 2)
    ix = ids[:, 0].reshape(M // C, C)
    iy = ids[:, 1].reshape(M // C, C) + POS
    idx2 = jnp.stack([ix, iy], axis=1).reshape(2 * M)
    pes = []
    tok0 = 0
    for mg in GROUPS:
        idx_g = lax.slice(idx2, (tok0 * 2,), ((tok0 + mg) * 2,))
        pes.append(_pe_gather_kernel(mg)(table2, idx_g))
        tok0 += mg
    h = None
    tok0 = 0
    for mg, pe_g in zip(GROUPS, pes):
        h = _mm_group(tok0, mg, px, W, pe_g, h)
        tok0 += mg
    return h.reshape(B, N, H)
